# baseline (speedup 1.0000x reference)
import functools

import jax
import jax.numpy as jnp
from jax import lax
from jax.experimental import pallas as pl
from jax.experimental.pallas import tpu as pltpu
from jax.experimental.pallas import tpu_sc as plsc

B, S = 128, 32768
NC, NS = 2, 16
NW = NC * NS
NT = 16
R8 = 8
BS = 2048
NBLK = S // BS
RT = 96
RSC = B - RT
NG = RSC // R8
TPG = NW // NG
CSC = S // TPG
CW = 2048
NCH = CSC // CW

_LOG1P_COEF = (
    1.0009290e-02, -5.2437536e-02, 1.3083343e-01, -2.2316587e-01,
    3.2722571e-01, -4.9928504e-01, 9.9996710e-01, 2.5546731e-07,
)
_C0 = 0.17328679513998632


def _pair_sc(s, pos):
    d = s - pos
    u = jnp.exp(jnp.minimum(d, -d))
    p = jnp.float32(_LOG1P_COEF[0])
    for c in _LOG1P_COEF[1:]:
        p = p * u + jnp.float32(c)
    t = jnp.maximum(d, 0.0) + p
    pt = jnp.exp(-t)
    w = 1.0 - pt
    return w * (w * t)


def _allsum(x, buf, iot):
    for k in (1, 2, 4, 8):
        buf[...] = x
        x = x + plsc.load_gather(buf, [jnp.bitwise_xor(iot, k)])
    return x


def _gather_body(scores_hbm, mask_hbm, head_hbm, tail_hbm, out_hbm,
                 hbuf, tbuf, wblkf, wblki, redbuf, obuf, gsem):
    wid = lax.axis_index("c") * NS + lax.axis_index("s")
    iot = lax.broadcasted_iota(jnp.int32, (16,), 0)

    @pl.when(wid < NT)
    def _():
        g8 = pl.multiple_of(wid * R8, 8)

        loff = pl.multiple_of(jnp.minimum(g8, B - 16), 8)
        lanebase = g8 - loff
        pltpu.sync_copy(head_hbm.at[pl.ds(loff, 16)], hbuf)
        pltpu.sync_copy(tail_hbm.at[pl.ds(loff, 16)], tbuf)
        hv = hbuf[...]
        tv = tbuf[...]

        tail_bs, head_bs, toffs, hoffs = [], [], [], []
        for rr in range(R8):
            insel = iot == (lanebase + rr)
            tail_b = jnp.sum(jnp.where(insel, tv, 0))
            head_b = jnp.sum(jnp.where(insel, hv, 0))
            tail_bs.append(tail_b)
            head_bs.append(head_b)
            toffs.append(pl.multiple_of((tail_b // 128) * 128, 128))
            hoffs.append(pl.multiple_of((head_b // 128) * 128, 128))
        copies = []
        for rr in range(R8):
            rowsl = pl.ds(g8, 8)
            copies.append(pltpu.async_copy(
                scores_hbm.at[rowsl, pl.ds(toffs[rr], 128)],
                wblkf.at[2 * rr + 0], gsem))
            copies.append(pltpu.async_copy(
                scores_hbm.at[rowsl, pl.ds(hoffs[rr], 128)],
                wblkf.at[2 * rr + 1], gsem))
            copies.append(pltpu.async_copy(
                mask_hbm.at[rowsl, pl.ds(toffs[rr], 128)],
                wblki.at[2 * rr + 0], gsem))
            copies.append(pltpu.async_copy(
                mask_hbm.at[rowsl, pl.ds(hoffs[rr], 128)],
                wblki.at[2 * rr + 1], gsem))
        for cp in copies:
            cp.wait()

        posp = jnp.zeros((16,), jnp.float32)
        csp = jnp.zeros((16,), jnp.float32)
        ccp = jnp.zeros((16,), jnp.float32)
        for rr in range(R8):
            ctl = tail_bs[rr] - toffs[rr]
            chl = head_bs[rr] - hoffs[rr]
            tsl = pl.ds(pl.multiple_of((ctl // 16) * 16, 16), 16)
            hsl = pl.ds(pl.multiple_of((chl // 16) * 16, 16), 16)
            sv = wblkf[2 * rr + 0, rr, tsl]
            posvec = _allsum(jnp.where(iot == ctl % 16, sv, 0.0),
                             redbuf, iot)
            shv = wblkf[2 * rr + 1, rr, hsl]
            shvec = _allsum(jnp.where(iot == chl % 16, shv, 0.0),
                            redbuf, iot)
            mtv = wblki[2 * rr + 0, rr, tsl]
            mt_vec = _allsum(
                jnp.where(iot == ctl % 16, mtv, 0).astype(jnp.float32),
                redbuf, iot)
            mhv = wblki[2 * rr + 1, rr, hsl]
            mh_vec = _allsum(
                jnp.where(iot == chl % 16, mhv, 0).astype(jnp.float32),
                redbuf, iot)
            hnev = jnp.where(
                jnp.full((16,), head_bs[rr], jnp.int32)
                != jnp.full((16,), tail_bs[rr], jnp.int32), 1.0, 0.0)
            lh = _pair_sc(shvec, posvec)
            cs = mh_vec * lh + hnev * mt_vec * jnp.float32(_C0)
            cc = mh_vec + hnev * mt_vec
            posp = jnp.where(iot == rr, posvec, posp)
            csp = jnp.where(iot == rr, cs, csp)
            ccp = jnp.where(iot == rr, cc, ccp)

        obuf[...] = posp
        pltpu.sync_copy(obuf, out_hbm.at[0, wid])
        obuf[...] = csp
        pltpu.sync_copy(obuf, out_hbm.at[1, wid])
        obuf[...] = ccp
        pltpu.sync_copy(obuf, out_hbm.at[2, wid])


def _scdense_body(scores_hbm, mask_hbm, g_hbm, out_hbm,
                  sbufs0, sbufs1, mbufs0, mbufs1, pbuf, cbuf, ccbuf,
                  redbuf, stage, stage2, obuf, shared, sems):
    wid = lax.axis_index("c") * NS + lax.axis_index("s")
    sid = lax.axis_index("s")
    iot = lax.broadcasted_iota(jnp.int32, (16,), 0)
    sbufs = (sbufs0, sbufs1)
    mbufs = (mbufs0, mbufs1)

    grp = wid // TPG
    part = wid % TPG
    rowbase = pl.multiple_of(RT + grp * R8, 8)
    colbase = part * CSC

    gtile = RT // R8 + grp
    pltpu.sync_copy(g_hbm.at[0, gtile], pbuf)
    pltpu.sync_copy(g_hbm.at[1, gtile], cbuf)
    pltpu.sync_copy(g_hbm.at[2, gtile], ccbuf)
    posv16 = pbuf[...]
    posvecs = [_allsum(jnp.where(iot == rr, posv16, 0.0), redbuf, iot)
               for rr in range(R8)]

    def start(c):
        buf = c % 2
        cb = pl.multiple_of(colbase + c * CW, 128)
        hs = pltpu.async_copy(
            scores_hbm.at[pl.ds(rowbase, 8), pl.ds(cb, CW)],
            sbufs[buf], sems[buf])
        hm = pltpu.async_copy(
            mask_hbm.at[pl.ds(rowbase, 8), pl.ds(cb, CW)],
            mbufs[buf], sems[2 + buf])
        return hs, hm

    accs = [jnp.zeros((16,), jnp.float32) for _ in range(2 * R8)]
    pending = start(0)
    for c in range(NCH):
        hs, hm = pending
        if c + 1 < NCH:
            nxt = start(c + 1)
        hs.wait()
        hm.wait()
        if c + 1 < NCH:
            pending = nxt
        sb = sbufs[c % 2]
        mb = mbufs[c % 2]

        def step(k, carry, sb=sb, mb=mb, posvecs=posvecs):
            carry = list(carry)
            off = k * 16
            for rr in range(R8):
                svec = sb[rr, pl.ds(off, 16)]
                mvec = mb[rr, pl.ds(off, 16)]
                mf = mvec.astype(jnp.float32)
                carry[rr] = carry[rr] + mf * _pair_sc(svec, posvecs[rr])
                carry[R8 + rr] = carry[R8 + rr] + mf
            return tuple(carry)

        accs = lax.fori_loop(0, CW // 16, step, tuple(accs))

    packs = jnp.zeros((16,), jnp.float32)
    packc = jnp.zeros((16,), jnp.float32)
    for rr in range(R8):
        rs = _allsum(accs[rr], redbuf, iot)
        rc = _allsum(accs[R8 + rr], redbuf, iot)
        packs = jnp.where(iot == rr, rs, packs)
        packc = jnp.where(iot == rr, rc, packc)

    obuf[...] = packs
    pltpu.sync_copy(obuf, out_hbm.at[0, wid])
    obuf[...] = packc
    pltpu.sync_copy(obuf, out_hbm.at[1, wid])


def _dense_body(pos_ref, cs_ref, cc_ref, scores_ref, mask_ref, out_ref,
                acc_s, acc_c):
    i = pl.program_id(0)

    @pl.when(i == 0)
    def _():
        acc_s[...] = jnp.zeros_like(acc_s)
        acc_c[...] = jnp.zeros_like(acc_c)

    s = scores_ref[...]
    m = mask_ref[...].astype(jnp.float32)
    d = s - pos_ref[...]
    u = jnp.exp(-jnp.abs(d))
    t = jnp.maximum(d, 0.0) + jnp.log1p(u)
    r = 1.0 / (1.0 + u)
    w = jnp.where(d > 0, r, u * r)
    acc_s[...] += m * (w * (w * t))
    acc_c[...] += m

    @pl.when(i == NBLK - 1)
    def _():
        rs = acc_s[...].sum(axis=1, keepdims=True) - cs_ref[...]
        rc = acc_c[...].sum(axis=1, keepdims=True) - cc_ref[...]
        rl = jnp.where(rc > 0.5, rs / jnp.maximum(rc, 1.0), 0.0)
        out_ref[...] = jnp.sum(rl).reshape(1, 1)


@jax.jit
def kernel(scores, head_position, tail_position, score_mask):
    mask = score_mask.astype(jnp.int32)
    head = head_position.astype(jnp.int32).reshape(B)
    tail = tail_position.astype(jnp.int32).reshape(B)

    gather = pl.kernel(
        _gather_body,
        out_type=jax.ShapeDtypeStruct((3, NT, 16), jnp.float32),
        mesh=plsc.VectorSubcoreMesh(core_axis_name="c", subcore_axis_name="s",
                                    num_cores=NC, num_subcores=NS),
        compiler_params=pltpu.CompilerParams(needs_layout_passes=False),
        scratch_types=[
            pltpu.VMEM((16,), jnp.int32),
            pltpu.VMEM((16,), jnp.int32),
            pltpu.VMEM((16, R8, 128), jnp.float32),
            pltpu.VMEM((16, R8, 128), jnp.int32),
            pltpu.VMEM((16,), jnp.float32),
            pltpu.VMEM((16,), jnp.float32),
            pltpu.SemaphoreType.DMA,
        ],
    )
    g = gather(scores, mask, head, tail)
    percol = g[:, :, :R8].reshape(3, B, 1)
    pos, cs, cc = percol[0], percol[1], percol[2]

    scdense = pl.kernel(
        _scdense_body,
        out_type=jax.ShapeDtypeStruct((2, NW, 16), jnp.float32),
        mesh=plsc.VectorSubcoreMesh(core_axis_name="c", subcore_axis_name="s",
                                    num_cores=NC, num_subcores=NS),
        compiler_params=pltpu.CompilerParams(needs_layout_passes=False),
        scratch_types=[
            pltpu.VMEM((R8, CW), jnp.float32),
            pltpu.VMEM((R8, CW), jnp.float32),
            pltpu.VMEM((R8, CW), jnp.int32),
            pltpu.VMEM((R8, CW), jnp.int32),
            pltpu.VMEM((16,), jnp.float32),
            pltpu.VMEM((16,), jnp.float32),
            pltpu.VMEM((16,), jnp.float32),
            pltpu.VMEM((16,), jnp.float32),
            pltpu.VMEM((32,), jnp.float32),
            pltpu.VMEM((32,), jnp.float32),
            pltpu.VMEM((16,), jnp.float32),
            pltpu.VMEM_SHARED((16, 32), jnp.float32),
            [pltpu.SemaphoreType.DMA] * 4,
        ],
    )
    scp = scdense(scores, mask, g)
    ps = scp[0].reshape(NG, TPG, 16)[:, :, :R8].sum(axis=1).reshape(RSC)
    pc = scp[1].reshape(NG, TPG, 16)[:, :, :R8].sum(axis=1).reshape(RSC)
    ts_ = ps - cs[RT:, 0]
    tc_ = pc - cc[RT:, 0]
    sc_rl = jnp.where(tc_ > 0.5, ts_ / jnp.maximum(tc_, 1.0), 0.0)

    dense = pl.pallas_call(
        _dense_body,
        grid=(NBLK,),
        in_specs=[
            pl.BlockSpec((RT, 1), lambda i: (0, 0)),
            pl.BlockSpec((RT, 1), lambda i: (0, 0)),
            pl.BlockSpec((RT, 1), lambda i: (0, 0)),
            pl.BlockSpec((RT, BS), lambda i: (0, i)),
            pl.BlockSpec((RT, BS), lambda i: (0, i)),
        ],
        out_specs=pl.BlockSpec((1, 1), lambda i: (0, 0)),
        out_shape=jax.ShapeDtypeStruct((1, 1), jnp.float32),
        scratch_shapes=[
            pltpu.VMEM((RT, BS), jnp.float32),
            pltpu.VMEM((RT, BS), jnp.float32),
        ],
        compiler_params=pltpu.CompilerParams(
            dimension_semantics=("arbitrary",)),
    )
    tc_sum = dense(pos[:RT], cs[:RT], cc[:RT], scores, mask)[0, 0]
    return (tc_sum + jnp.sum(sc_rl)) * jnp.float32(1.0 / B)


# speedup vs baseline: 1.0478x; 1.0478x over previous
import functools

import jax
import jax.numpy as jnp
from jax import lax
from jax.experimental import pallas as pl
from jax.experimental.pallas import tpu as pltpu
from jax.experimental.pallas import tpu_sc as plsc

B, S = 128, 32768
NC, NS = 2, 16
NW = NC * NS
NT = 16
R8 = 8
BS = 2048
NBLK = S // BS
RT = 96
RSC = B - RT
NG = RSC // R8
TPG = NW // NG
CSC = S // TPG
CW = 2048
NCH = CSC // CW

_LOG1P_COEF = (
    1.0009290e-02, -5.2437536e-02, 1.3083343e-01, -2.2316587e-01,
    3.2722571e-01, -4.9928504e-01, 9.9996710e-01, 2.5546731e-07,
)
_C0 = 0.17328679513998632


def _pair_sc(s, pos):
    d = s - pos
    u = jnp.exp(jnp.minimum(d, -d))
    p = jnp.float32(_LOG1P_COEF[0])
    for c in _LOG1P_COEF[1:]:
        p = p * u + jnp.float32(c)
    t = jnp.maximum(d, 0.0) + p
    pt = jnp.exp(-t)
    w = 1.0 - pt
    return w * (w * t)


def _allsum(x, buf, iot):
    for k in (1, 2, 4, 8):
        buf[...] = x
        x = x + plsc.load_gather(buf, [jnp.bitwise_xor(iot, k)])
    return x


def _posgather_body(scores_hbm, tail_hbm, out_hbm,
                    tbuf, wblkf, redbuf, obuf, gsem):
    wid = lax.axis_index("c") * NS + lax.axis_index("s")
    iot = lax.broadcasted_iota(jnp.int32, (16,), 0)

    @pl.when(wid < NT)
    def _():
        g8 = pl.multiple_of(wid * R8, 8)
        loff = pl.multiple_of(jnp.minimum(g8, B - 16), 8)
        lanebase = g8 - loff
        pltpu.sync_copy(tail_hbm.at[pl.ds(loff, 16)], tbuf)
        tv = tbuf[...]

        tail_bs, toffs = [], []
        for rr in range(R8):
            tail_b = jnp.sum(jnp.where(iot == (lanebase + rr), tv, 0))
            tail_bs.append(tail_b)
            toffs.append(pl.multiple_of((tail_b // 128) * 128, 128))
        copies = []
        for rr in range(R8):
            copies.append(pltpu.async_copy(
                scores_hbm.at[pl.ds(g8, 8), pl.ds(toffs[rr], 128)],
                wblkf.at[rr], gsem))
        for cp in copies:
            cp.wait()

        posp = jnp.zeros((16,), jnp.float32)
        for rr in range(R8):
            ctl = tail_bs[rr] - toffs[rr]
            tsl = pl.ds(pl.multiple_of((ctl // 16) * 16, 16), 16)
            sv = wblkf[rr, rr, tsl]
            posvec = _allsum(jnp.where(iot == ctl % 16, sv, 0.0),
                             redbuf, iot)
            posp = jnp.where(iot == rr, posvec, posp)

        obuf[...] = posp
        pltpu.sync_copy(obuf, out_hbm.at[wid])


def _scdense_body(scores_hbm, mask_hbm, head_hbm, tail_hbm, pos_hbm,
                  out_hbm, cor_hbm,
                  sbufs0, sbufs1, mbufs0, mbufs1, hbuf, tbuf, pbuf,
                  wblkf, wblki, redbuf, obuf, sems, gsem):
    wid = lax.axis_index("c") * NS + lax.axis_index("s")
    iot = lax.broadcasted_iota(jnp.int32, (16,), 0)
    sbufs = (sbufs0, sbufs1)
    mbufs = (mbufs0, mbufs1)

    grp = wid // TPG
    part = wid % TPG
    rowbase = pl.multiple_of(RT + grp * R8, 8)
    colbase = part * CSC

    def start(c):
        buf = c % 2
        cb = pl.multiple_of(colbase + c * CW, 128)
        hs = pltpu.async_copy(
            scores_hbm.at[pl.ds(rowbase, 8), pl.ds(cb, CW)],
            sbufs[buf], sems[buf])
        hm = pltpu.async_copy(
            mask_hbm.at[pl.ds(rowbase, 8), pl.ds(cb, CW)],
            mbufs[buf], sems[2 + buf])
        return hs, hm

    pending = start(0)

    pltpu.sync_copy(pos_hbm.at[RT // R8 + grp], pbuf)
    posv16 = pbuf[...]
    posvecs = [_allsum(jnp.where(iot == rr, posv16, 0.0), redbuf, iot)
               for rr in range(R8)]

    @pl.when(wid < NT)
    def _():
        g8 = pl.multiple_of(wid * R8, 8)
        loff = pl.multiple_of(jnp.minimum(g8, B - 16), 8)
        lanebase = g8 - loff
        pltpu.sync_copy(head_hbm.at[pl.ds(loff, 16)], hbuf)
        pltpu.sync_copy(tail_hbm.at[pl.ds(loff, 16)], tbuf)
        pltpu.sync_copy(pos_hbm.at[wid], pbuf)
        hv = hbuf[...]
        tv = tbuf[...]
        mypos = pbuf[...]

        tail_bs, head_bs, toffs, hoffs = [], [], [], []
        for rr in range(R8):
            insel = iot == (lanebase + rr)
            tail_b = jnp.sum(jnp.where(insel, tv, 0))
            head_b = jnp.sum(jnp.where(insel, hv, 0))
            tail_bs.append(tail_b)
            head_bs.append(head_b)
            toffs.append(pl.multiple_of((tail_b // 128) * 128, 128))
            hoffs.append(pl.multiple_of((head_b // 128) * 128, 128))
        copies = []
        for rr in range(R8):
            rowsl = pl.ds(g8, 8)
            copies.append(pltpu.async_copy(
                scores_hbm.at[rowsl, pl.ds(hoffs[rr], 128)],
                wblkf.at[rr], gsem))
            copies.append(pltpu.async_copy(
                mask_hbm.at[rowsl, pl.ds(toffs[rr], 128)],
                wblki.at[2 * rr + 0], gsem))
            copies.append(pltpu.async_copy(
                mask_hbm.at[rowsl, pl.ds(hoffs[rr], 128)],
                wblki.at[2 * rr + 1], gsem))
        for cp in copies:
            cp.wait()

        csp = jnp.zeros((16,), jnp.float32)
        ccp = jnp.zeros((16,), jnp.float32)
        for rr in range(R8):
            ctl = tail_bs[rr] - toffs[rr]
            chl = head_bs[rr] - hoffs[rr]
            tsl = pl.ds(pl.multiple_of((ctl // 16) * 16, 16), 16)
            hsl = pl.ds(pl.multiple_of((chl // 16) * 16, 16), 16)
            posvec = _allsum(jnp.where(iot == rr, mypos, 0.0),
                             redbuf, iot)
            shv = wblkf[rr, rr, hsl]
            shvec = _allsum(jnp.where(iot == chl % 16, shv, 0.0),
                            redbuf, iot)
            mtv = wblki[2 * rr + 0, rr, tsl]
            mt_vec = _allsum(
                jnp.where(iot == ctl % 16, mtv, 0).astype(jnp.float32),
                redbuf, iot)
            mhv = wblki[2 * rr + 1, rr, hsl]
            mh_vec = _allsum(
                jnp.where(iot == chl % 16, mhv, 0).astype(jnp.float32),
                redbuf, iot)
            hnev = jnp.where(
                jnp.full((16,), head_bs[rr], jnp.int32)
                != jnp.full((16,), tail_bs[rr], jnp.int32), 1.0, 0.0)
            lh = _pair_sc(shvec, posvec)
            cs = mh_vec * lh + hnev * mt_vec * jnp.float32(_C0)
            cc = mh_vec + hnev * mt_vec
            csp = jnp.where(iot == rr, cs, csp)
            ccp = jnp.where(iot == rr, cc, ccp)

        obuf[...] = csp
        pltpu.sync_copy(obuf, cor_hbm.at[0, wid])
        obuf[...] = ccp
        pltpu.sync_copy(obuf, cor_hbm.at[1, wid])

    accs = [jnp.zeros((16,), jnp.float32) for _ in range(2 * R8)]
    for c in range(NCH):
        hs, hm = pending
        if c + 1 < NCH:
            nxt = start(c + 1)
        hs.wait()
        hm.wait()
        if c + 1 < NCH:
            pending = nxt
        sb = sbufs[c % 2]
        mb = mbufs[c % 2]

        def step(k, carry, sb=sb, mb=mb, posvecs=posvecs):
            carry = list(carry)
            off = k * 16
            for rr in range(R8):
                svec = sb[rr, pl.ds(off, 16)]
                mvec = mb[rr, pl.ds(off, 16)]
                mf = mvec.astype(jnp.float32)
                carry[rr] = carry[rr] + mf * _pair_sc(svec, posvecs[rr])
                carry[R8 + rr] = carry[R8 + rr] + mf
            return tuple(carry)

        accs = lax.fori_loop(0, CW // 16, step, tuple(accs))

    packs = jnp.zeros((16,), jnp.float32)
    packc = jnp.zeros((16,), jnp.float32)
    for rr in range(R8):
        rs = _allsum(accs[rr], redbuf, iot)
        rc = _allsum(accs[R8 + rr], redbuf, iot)
        packs = jnp.where(iot == rr, rs, packs)
        packc = jnp.where(iot == rr, rc, packc)

    obuf[...] = packs
    pltpu.sync_copy(obuf, out_hbm.at[0, wid])
    obuf[...] = packc
    pltpu.sync_copy(obuf, out_hbm.at[1, wid])


def _dense_body(pos_ref, scores_ref, mask_ref, sum_ref, cnt_ref,
                acc_s, acc_c):
    i = pl.program_id(0)

    @pl.when(i == 0)
    def _():
        acc_s[...] = jnp.zeros_like(acc_s)
        acc_c[...] = jnp.zeros_like(acc_c)

    s = scores_ref[...]
    m = mask_ref[...].astype(jnp.float32)
    d = s - pos_ref[...]
    u = jnp.exp(-jnp.abs(d))
    t = jnp.maximum(d, 0.0) + jnp.log1p(u)
    r = 1.0 / (1.0 + u)
    w = jnp.where(d > 0, r, u * r)
    acc_s[...] += m * (w * (w * t))
    acc_c[...] += m

    @pl.when(i == NBLK - 1)
    def _():
        sum_ref[...] = acc_s[...].sum(axis=1, keepdims=True)
        cnt_ref[...] = acc_c[...].sum(axis=1, keepdims=True)


@jax.jit
def kernel(scores, head_position, tail_position, score_mask):
    mask = score_mask.astype(jnp.int32)
    head = head_position.astype(jnp.int32).reshape(B)
    tail = tail_position.astype(jnp.int32).reshape(B)

    posgather = pl.kernel(
        _posgather_body,
        out_type=jax.ShapeDtypeStruct((NT, 16), jnp.float32),
        mesh=plsc.VectorSubcoreMesh(core_axis_name="c", subcore_axis_name="s",
                                    num_cores=NC, num_subcores=NS),
        compiler_params=pltpu.CompilerParams(needs_layout_passes=False),
        scratch_types=[
            pltpu.VMEM((16,), jnp.int32),
            pltpu.VMEM((R8, R8, 128), jnp.float32),
            pltpu.VMEM((16,), jnp.float32),
            pltpu.VMEM((16,), jnp.float32),
            pltpu.SemaphoreType.DMA,
        ],
    )
    pg = posgather(scores, tail)
    pos = pg[:, :R8].reshape(B, 1)

    scdense = pl.kernel(
        _scdense_body,
        out_type=[jax.ShapeDtypeStruct((2, NW, 16), jnp.float32),
                  jax.ShapeDtypeStruct((2, NT, 16), jnp.float32)],
        mesh=plsc.VectorSubcoreMesh(core_axis_name="c", subcore_axis_name="s",
                                    num_cores=NC, num_subcores=NS),
        compiler_params=pltpu.CompilerParams(needs_layout_passes=False),
        scratch_types=[
            pltpu.VMEM((R8, CW), jnp.float32),
            pltpu.VMEM((R8, CW), jnp.float32),
            pltpu.VMEM((R8, CW), jnp.int32),
            pltpu.VMEM((R8, CW), jnp.int32),
            pltpu.VMEM((16,), jnp.int32),
            pltpu.VMEM((16,), jnp.int32),
            pltpu.VMEM((16,), jnp.float32),
            pltpu.VMEM((R8, R8, 128), jnp.float32),
            pltpu.VMEM((16, R8, 128), jnp.int32),
            pltpu.VMEM((16,), jnp.float32),
            pltpu.VMEM((16,), jnp.float32),
            [pltpu.SemaphoreType.DMA] * 4,
            pltpu.SemaphoreType.DMA,
        ],
    )
    scp, cor = scdense(scores, mask, head, tail, pg)

    dense = pl.pallas_call(
        _dense_body,
        grid=(NBLK,),
        in_specs=[
            pl.BlockSpec((RT, 1), lambda i: (0, 0)),
            pl.BlockSpec((RT, BS), lambda i: (0, i)),
            pl.BlockSpec((RT, BS), lambda i: (0, i)),
        ],
        out_specs=[
            pl.BlockSpec((RT, 1), lambda i: (0, 0)),
            pl.BlockSpec((RT, 1), lambda i: (0, 0)),
        ],
        out_shape=[jax.ShapeDtypeStruct((RT, 1), jnp.float32),
                   jax.ShapeDtypeStruct((RT, 1), jnp.float32)],
        scratch_shapes=[
            pltpu.VMEM((RT, BS), jnp.float32),
            pltpu.VMEM((RT, BS), jnp.float32),
        ],
        compiler_params=pltpu.CompilerParams(
            dimension_semantics=("arbitrary",)),
    )
    tc_sum, tc_cnt = dense(pos[:RT], scores, mask)

    cs = cor[0, :, :R8].reshape(B)
    cc = cor[1, :, :R8].reshape(B)
    sc_s = scp[0].reshape(NG, TPG, 16)[:, :, :R8].sum(axis=1).reshape(RSC)
    sc_c = scp[1].reshape(NG, TPG, 16)[:, :, :R8].sum(axis=1).reshape(RSC)
    rsum = jnp.concatenate([tc_sum[:, 0], sc_s]) - cs
    rcnt = jnp.concatenate([tc_cnt[:, 0], sc_c]) - cc
    rl = jnp.where(rcnt > 0.5, rsum / jnp.maximum(rcnt, 1.0), 0.0)
    return jnp.sum(rl) * jnp.float32(1.0 / B)


# speedup vs baseline: 1.0547x; 1.0066x over previous
import functools

import jax
import jax.numpy as jnp
from jax import lax
from jax.experimental import pallas as pl
from jax.experimental.pallas import tpu as pltpu
from jax.experimental.pallas import tpu_sc as plsc

B, S = 128, 32768
NC, NS = 2, 16
NW = NC * NS
NT = 16
R8 = 8
BS = 2048
KW = 256
NBLK = S // BS
RT = 96
RSC = B - RT
NG = RSC // R8
TPG = NW // NG
CSC = S // TPG
CW = 2048
NCH = CSC // CW

_LOG1P_COEF = (
    1.0009290e-02, -5.2437536e-02, 1.3083343e-01, -2.2316587e-01,
    3.2722571e-01, -4.9928504e-01, 9.9996710e-01, 2.5546731e-07,
)
_C0 = 0.17328679513998632


def _pair_sc(s, pos):
    d = s - pos
    u = jnp.exp(jnp.minimum(d, -d))
    p = jnp.float32(_LOG1P_COEF[0])
    for c in _LOG1P_COEF[1:]:
        p = p * u + jnp.float32(c)
    t = jnp.maximum(d, 0.0) + p
    pt = jnp.exp(-t)
    w = 1.0 - pt
    return w * (w * t)


def _allsum(x, buf, iot):
    for k in (1, 2, 4, 8):
        buf[...] = x
        x = x + plsc.load_gather(buf, [jnp.bitwise_xor(iot, k)])
    return x


def _posgather_body(scores_hbm, tail_hbm, out_hbm,
                    tbuf, wblkf, redbuf, obuf, gsem):
    wid = lax.axis_index("c") * NS + lax.axis_index("s")
    iot = lax.broadcasted_iota(jnp.int32, (16,), 0)

    @pl.when(wid < NT)
    def _():
        g8 = pl.multiple_of(wid * R8, 8)
        loff = pl.multiple_of(jnp.minimum(g8, B - 16), 8)
        lanebase = g8 - loff
        pltpu.sync_copy(tail_hbm.at[pl.ds(loff, 16)], tbuf)
        tv = tbuf[...]

        tail_bs, toffs = [], []
        for rr in range(R8):
            tail_b = jnp.sum(jnp.where(iot == (lanebase + rr), tv, 0))
            tail_bs.append(tail_b)
            toffs.append(pl.multiple_of((tail_b // 128) * 128, 128))
        copies = []
        for rr in range(R8):
            copies.append(pltpu.async_copy(
                scores_hbm.at[pl.ds(g8, 8), pl.ds(toffs[rr], 128)],
                wblkf.at[rr], gsem))
        for cp in copies:
            cp.wait()

        posp = jnp.zeros((16,), jnp.float32)
        for rr in range(R8):
            ctl = tail_bs[rr] - toffs[rr]
            tsl = pl.ds(pl.multiple_of((ctl // 16) * 16, 16), 16)
            sv = wblkf[rr, rr, tsl]
            posvec = _allsum(jnp.where(iot == ctl % 16, sv, 0.0),
                             redbuf, iot)
            posp = jnp.where(iot == rr, posvec, posp)

        obuf[...] = posp
        pltpu.sync_copy(obuf, out_hbm.at[wid])


def _scdense_body(scores_hbm, mask_hbm, head_hbm, tail_hbm, pos_hbm,
                  out_hbm, cor_hbm,
                  sbufs0, sbufs1, mbufs0, mbufs1, hbuf, tbuf, pbuf,
                  wblkf, wblki, redbuf, obuf, sems, gsem):
    wid = lax.axis_index("c") * NS + lax.axis_index("s")
    iot = lax.broadcasted_iota(jnp.int32, (16,), 0)
    sbufs = (sbufs0, sbufs1)
    mbufs = (mbufs0, mbufs1)

    grp = wid // TPG
    part = wid % TPG
    rowbase = pl.multiple_of(RT + grp * R8, 8)
    colbase = part * CSC

    def start(c):
        buf = c % 2
        cb = pl.multiple_of(colbase + c * CW, 128)
        hs = pltpu.async_copy(
            scores_hbm.at[pl.ds(rowbase, 8), pl.ds(cb, CW)],
            sbufs[buf], sems[buf])
        hm = pltpu.async_copy(
            mask_hbm.at[pl.ds(rowbase, 8), pl.ds(cb, CW)],
            mbufs[buf], sems[2 + buf])
        return hs, hm

    pending = start(0)

    pltpu.sync_copy(pos_hbm.at[RT // R8 + grp], pbuf)
    posv16 = pbuf[...]
    posvecs = [_allsum(jnp.where(iot == rr, posv16, 0.0), redbuf, iot)
               for rr in range(R8)]

    @pl.when(wid < NT)
    def _():
        g8 = pl.multiple_of(wid * R8, 8)
        loff = pl.multiple_of(jnp.minimum(g8, B - 16), 8)
        lanebase = g8 - loff
        pltpu.sync_copy(head_hbm.at[pl.ds(loff, 16)], hbuf)
        pltpu.sync_copy(tail_hbm.at[pl.ds(loff, 16)], tbuf)
        pltpu.sync_copy(pos_hbm.at[wid], pbuf)
        hv = hbuf[...]
        tv = tbuf[...]
        mypos = pbuf[...]

        tail_bs, head_bs, toffs, hoffs = [], [], [], []
        for rr in range(R8):
            insel = iot == (lanebase + rr)
            tail_b = jnp.sum(jnp.where(insel, tv, 0))
            head_b = jnp.sum(jnp.where(insel, hv, 0))
            tail_bs.append(tail_b)
            head_bs.append(head_b)
            toffs.append(pl.multiple_of((tail_b // 128) * 128, 128))
            hoffs.append(pl.multiple_of((head_b // 128) * 128, 128))
        copies = []
        for rr in range(R8):
            rowsl = pl.ds(g8, 8)
            copies.append(pltpu.async_copy(
                scores_hbm.at[rowsl, pl.ds(hoffs[rr], 128)],
                wblkf.at[rr], gsem))
            copies.append(pltpu.async_copy(
                mask_hbm.at[rowsl, pl.ds(toffs[rr], 128)],
                wblki.at[2 * rr + 0], gsem))
            copies.append(pltpu.async_copy(
                mask_hbm.at[rowsl, pl.ds(hoffs[rr], 128)],
                wblki.at[2 * rr + 1], gsem))
        for cp in copies:
            cp.wait()

        csp = jnp.zeros((16,), jnp.float32)
        ccp = jnp.zeros((16,), jnp.float32)
        for rr in range(R8):
            ctl = tail_bs[rr] - toffs[rr]
            chl = head_bs[rr] - hoffs[rr]
            tsl = pl.ds(pl.multiple_of((ctl // 16) * 16, 16), 16)
            hsl = pl.ds(pl.multiple_of((chl // 16) * 16, 16), 16)
            posvec = _allsum(jnp.where(iot == rr, mypos, 0.0),
                             redbuf, iot)
            shv = wblkf[rr, rr, hsl]
            shvec = _allsum(jnp.where(iot == chl % 16, shv, 0.0),
                            redbuf, iot)
            mtv = wblki[2 * rr + 0, rr, tsl]
            mt_vec = _allsum(
                jnp.where(iot == ctl % 16, mtv, 0).astype(jnp.float32),
                redbuf, iot)
            mhv = wblki[2 * rr + 1, rr, hsl]
            mh_vec = _allsum(
                jnp.where(iot == chl % 16, mhv, 0).astype(jnp.float32),
                redbuf, iot)
            hnev = jnp.where(
                jnp.full((16,), head_bs[rr], jnp.int32)
                != jnp.full((16,), tail_bs[rr], jnp.int32), 1.0, 0.0)
            lh = _pair_sc(shvec, posvec)
            cs = mh_vec * lh + hnev * mt_vec * jnp.float32(_C0)
            cc = mh_vec + hnev * mt_vec
            csp = jnp.where(iot == rr, cs, csp)
            ccp = jnp.where(iot == rr, cc, ccp)

        obuf[...] = csp
        pltpu.sync_copy(obuf, cor_hbm.at[0, wid])
        obuf[...] = ccp
        pltpu.sync_copy(obuf, cor_hbm.at[1, wid])

    accs = [jnp.zeros((16,), jnp.float32) for _ in range(2 * R8)]
    for c in range(NCH):
        hs, hm = pending
        if c + 1 < NCH:
            nxt = start(c + 1)
        hs.wait()
        hm.wait()
        if c + 1 < NCH:
            pending = nxt
        sb = sbufs[c % 2]
        mb = mbufs[c % 2]

        def step(k, carry, sb=sb, mb=mb, posvecs=posvecs):
            carry = list(carry)
            off = k * 16
            for rr in range(R8):
                svec = sb[rr, pl.ds(off, 16)]
                mvec = mb[rr, pl.ds(off, 16)]
                mf = mvec.astype(jnp.float32)
                carry[rr] = carry[rr] + mf * _pair_sc(svec, posvecs[rr])
                carry[R8 + rr] = carry[R8 + rr] + mf
            return tuple(carry)

        accs = lax.fori_loop(0, CW // 16, step, tuple(accs))

    packs = jnp.zeros((16,), jnp.float32)
    packc = jnp.zeros((16,), jnp.float32)
    for rr in range(R8):
        rs = _allsum(accs[rr], redbuf, iot)
        rc = _allsum(accs[R8 + rr], redbuf, iot)
        packs = jnp.where(iot == rr, rs, packs)
        packc = jnp.where(iot == rr, rc, packc)

    obuf[...] = packs
    pltpu.sync_copy(obuf, out_hbm.at[0, wid])
    obuf[...] = packc
    pltpu.sync_copy(obuf, out_hbm.at[1, wid])


def _dense_body(pos_ref, scores_ref, mask_ref, sum_ref, cnt_ref,
                acc_s, acc_c):
    i = pl.program_id(0)

    @pl.when(i == 0)
    def _():
        acc_s[...] = jnp.zeros_like(acc_s)
        acc_c[...] = jnp.zeros_like(acc_c)

    pos = pos_ref[...]
    for k in range(BS // KW):
        sl = pl.ds(k * KW, KW)
        s = scores_ref[:, sl]
        m = mask_ref[:, sl].astype(jnp.float32)
        d = s - pos
        u = jnp.exp(-jnp.abs(d))
        t = jnp.maximum(d, 0.0) + jnp.log1p(u)
        r = 1.0 / (1.0 + u)
        w = jnp.where(d > 0, r, u * r)
        acc_s[:, sl] += m * (w * (w * t))
        acc_c[:, sl] += m

    @pl.when(i == NBLK - 1)
    def _():
        sum_ref[...] = acc_s[...].sum(axis=1, keepdims=True)
        cnt_ref[...] = acc_c[...].sum(axis=1, keepdims=True)


@jax.jit
def kernel(scores, head_position, tail_position, score_mask):
    mask = score_mask.astype(jnp.int32)
    head = head_position.astype(jnp.int32).reshape(B)
    tail = tail_position.astype(jnp.int32).reshape(B)

    posgather = pl.kernel(
        _posgather_body,
        out_type=jax.ShapeDtypeStruct((NT, 16), jnp.float32),
        mesh=plsc.VectorSubcoreMesh(core_axis_name="c", subcore_axis_name="s",
                                    num_cores=NC, num_subcores=NS),
        compiler_params=pltpu.CompilerParams(needs_layout_passes=False),
        scratch_types=[
            pltpu.VMEM((16,), jnp.int32),
            pltpu.VMEM((R8, R8, 128), jnp.float32),
            pltpu.VMEM((16,), jnp.float32),
            pltpu.VMEM((16,), jnp.float32),
            pltpu.SemaphoreType.DMA,
        ],
    )
    pg = posgather(scores, tail)
    pos = pg[:, :R8].reshape(B, 1)

    scdense = pl.kernel(
        _scdense_body,
        out_type=[jax.ShapeDtypeStruct((2, NW, 16), jnp.float32),
                  jax.ShapeDtypeStruct((2, NT, 16), jnp.float32)],
        mesh=plsc.VectorSubcoreMesh(core_axis_name="c", subcore_axis_name="s",
                                    num_cores=NC, num_subcores=NS),
        compiler_params=pltpu.CompilerParams(needs_layout_passes=False),
        scratch_types=[
            pltpu.VMEM((R8, CW), jnp.float32),
            pltpu.VMEM((R8, CW), jnp.float32),
            pltpu.VMEM((R8, CW), jnp.int32),
            pltpu.VMEM((R8, CW), jnp.int32),
            pltpu.VMEM((16,), jnp.int32),
            pltpu.VMEM((16,), jnp.int32),
            pltpu.VMEM((16,), jnp.float32),
            pltpu.VMEM((R8, R8, 128), jnp.float32),
            pltpu.VMEM((16, R8, 128), jnp.int32),
            pltpu.VMEM((16,), jnp.float32),
            pltpu.VMEM((16,), jnp.float32),
            [pltpu.SemaphoreType.DMA] * 4,
            pltpu.SemaphoreType.DMA,
        ],
    )
    scp, cor = scdense(scores, mask, head, tail, pg)

    dense = pl.pallas_call(
        _dense_body,
        grid=(NBLK,),
        in_specs=[
            pl.BlockSpec((RT, 1), lambda i: (0, 0)),
            pl.BlockSpec((RT, BS), lambda i: (0, i)),
            pl.BlockSpec((RT, BS), lambda i: (0, i)),
        ],
        out_specs=[
            pl.BlockSpec((RT, 1), lambda i: (0, 0)),
            pl.BlockSpec((RT, 1), lambda i: (0, 0)),
        ],
        out_shape=[jax.ShapeDtypeStruct((RT, 1), jnp.float32),
                   jax.ShapeDtypeStruct((RT, 1), jnp.float32)],
        scratch_shapes=[
            pltpu.VMEM((RT, BS), jnp.float32),
            pltpu.VMEM((RT, BS), jnp.float32),
        ],
        compiler_params=pltpu.CompilerParams(
            dimension_semantics=("arbitrary",)),
    )
    tc_sum, tc_cnt = dense(pos[:RT], scores, mask)

    cs = cor[0, :, :R8].reshape(B)
    cc = cor[1, :, :R8].reshape(B)
    sc_s = scp[0].reshape(NG, TPG, 16)[:, :, :R8].sum(axis=1).reshape(RSC)
    sc_c = scp[1].reshape(NG, TPG, 16)[:, :, :R8].sum(axis=1).reshape(RSC)
    rsum = jnp.concatenate([tc_sum[:, 0], sc_s]) - cs
    rcnt = jnp.concatenate([tc_cnt[:, 0], sc_c]) - cc
    rl = jnp.where(rcnt > 0.5, rsum / jnp.maximum(rcnt, 1.0), 0.0)
    return jnp.sum(rl) * jnp.float32(1.0 / B)
